# baseline (speedup 1.0000x reference)
import functools

import jax
import jax.numpy as jnp
from jax import lax
from jax.experimental import pallas as pl
from jax.experimental.pallas import tpu as pltpu
from jax.experimental.pallas import tpu_sc as plsc

N_TOKENS = 16384
N_CODES = 8192
DIM = 256
COMMIT = 0.25

BM = 512
NSTEPS = N_TOKENS // BM

_KEY_BASE = 0x4B000000


def _dist_argmin_body(z_ref, e_ref, key_ref, idx_ref, loss_ref,
                      mm_s, zsq_s, acc_ref):
    m = pl.program_id(0)
    par = lax.rem(m, 2)
    zb = z_ref[...]
    eb = e_ref[...]
    mm_s[par] = lax.dot_general(zb, eb, (((1,), (1,)), ((), ())),
                                preferred_element_type=jnp.float32)
    zsq_s[par] = jnp.sum(zb * zb, axis=1, keepdims=True)

    prev = 1 - par
    mm = mm_s[prev]
    zsq = zsq_s[prev]
    mmax = jnp.max(mm, axis=1, keepdims=True)
    dmin = zsq - 2.0 * mmax
    db = lax.bitcast_convert_type(dmin, jnp.int32)
    dnext = lax.bitcast_convert_type(db + 1, jnp.float32)
    ulp = dnext - dmin
    h = ((zsq - dmin) - 0.5 * ulp) * 0.5
    hb = lax.bitcast_convert_type(h, jnp.int32)
    hup = lax.bitcast_convert_type(
        jnp.where(h >= 0.0, hb + 1, hb - 1), jnp.float32)
    thr = jnp.where((db & 1) == 0, h, hup)
    key = key_ref[...]
    sel = jnp.where(mm >= thr, key, jnp.float32(2.0 ** 24))
    kmin = jnp.min(sel, axis=1, keepdims=True)
    idx_ref[...] = lax.bitcast_convert_type(kmin, jnp.int32) - _KEY_BASE

    @pl.when(m == 1)
    def _init():
        acc_ref[0] = 0.0

    @pl.when(m >= 1)
    def _acc():
        acc_ref[0] += jnp.sum(dmin)

    @pl.when(m == NSTEPS)
    def _fin():
        scale = (1.0 + COMMIT) / (N_TOKENS * DIM)
        loss_ref[...] = jnp.broadcast_to(acc_ref[0] * scale, (1, 1))


def _distances_argmin(z, embedding_weight):
    keys = lax.bitcast_convert_type(
        jnp.arange(N_CODES, dtype=jnp.int32) + jnp.int32(_KEY_BASE),
        jnp.float32).reshape(1, N_CODES)
    return pl.pallas_call(
        _dist_argmin_body,
        grid=(NSTEPS + 1,),
        in_specs=[
            pl.BlockSpec((BM, DIM), lambda m: (jnp.minimum(m, NSTEPS - 1), 0)),
            pl.BlockSpec((N_CODES, DIM), lambda m: (0, 0)),
            pl.BlockSpec((1, N_CODES), lambda m: (0, 0)),
        ],
        out_specs=[
            pl.BlockSpec((BM, 1), lambda m: (jnp.maximum(m - 1, 0), 0)),
            pl.BlockSpec((1, 1), lambda m: (0, 0)),
        ],
        out_shape=[
            jax.ShapeDtypeStruct((N_TOKENS, 1), jnp.int32),
            jax.ShapeDtypeStruct((1, 1), jnp.float32),
        ],
        scratch_shapes=[
            pltpu.VMEM((2, BM, N_CODES), jnp.float32),
            pltpu.VMEM((2, BM, 1), jnp.float32),
            pltpu.SMEM((1,), jnp.float32),
        ],
        compiler_params=pltpu.CompilerParams(
            dimension_semantics=("arbitrary",)),
    )(z, embedding_weight, keys)


def _sc_gather(table, idx):
    info = plsc.get_sparse_core_info()
    nw = info.num_cores * info.num_subcores
    b_per_w = N_TOKENS // nw
    ch = 128
    nch = b_per_w // ch
    mesh = plsc.VectorSubcoreMesh(core_axis_name="c", subcore_axis_name="s")

    @functools.partial(
        pl.kernel, mesh=mesh,
        out_type=jax.ShapeDtypeStruct((N_TOKENS, DIM), jnp.float32),
        scratch_types=[
            pltpu.VMEM((b_per_w,), jnp.int32),
            pltpu.VMEM((ch, DIM), jnp.float32),
            pltpu.VMEM((ch, DIM), jnp.float32),
            pltpu.SemaphoreType.DMA,
            pltpu.SemaphoreType.DMA,
        ],
    )
    def k(table_hbm, idx_hbm, out_hbm, idx_v, buf0, buf1, sem0, sem1):
        wid = lax.axis_index("s") * info.num_cores + lax.axis_index("c")
        base = wid * b_per_w
        pltpu.sync_copy(idx_hbm.at[pl.ds(base, b_per_w)], idx_v)
        bufs = (buf0, buf1)
        sems = (sem0, sem1)
        copies = [None, None]
        copies[0] = pltpu.async_copy(
            table_hbm.at[idx_v.at[pl.ds(0, ch)]], bufs[0], sems[0])
        for c in range(nch):
            if c + 1 < nch:
                copies[(c + 1) % 2] = pltpu.async_copy(
                    table_hbm.at[idx_v.at[pl.ds((c + 1) * ch, ch)]],
                    bufs[(c + 1) % 2], sems[(c + 1) % 2])
            copies[c % 2].wait()
            pltpu.sync_copy(bufs[c % 2], out_hbm.at[pl.ds(base + c * ch, ch)])

    return k(table, idx)


def kernel(z, embedding_weight):
    z_flat = z.reshape(N_TOKENS, DIM)
    idx2d, loss2d = _distances_argmin(z_flat, embedding_weight)
    code_indices = idx2d.reshape(N_TOKENS)
    quantized_st = _sc_gather(embedding_weight, code_indices)
    loss = loss2d.reshape(())
    return (quantized_st, loss, code_indices)


# speedup vs baseline: 1.5289x; 1.5289x over previous
import functools

import jax
import jax.numpy as jnp
from jax import lax
from jax.experimental import pallas as pl
from jax.experimental.pallas import tpu as pltpu
from jax.experimental.pallas import tpu_sc as plsc

N_TOKENS = 16384
N_CODES = 8192
DIM = 256
COMMIT = 0.25

BM = 1024
NSTEPS = N_TOKENS // BM

_KEY_BASE = 0x4B000000


def _dist_argmin_body(z_ref, e_ref, key_ref, idx_ref, loss_ref, acc_ref):
    m = pl.program_id(0)
    zb = z_ref[...]
    eb = e_ref[...]
    mm = lax.dot_general(zb, eb, (((1,), (1,)), ((), ())),
                         preferred_element_type=jnp.float32)
    zsq = jnp.sum(zb * zb, axis=1, keepdims=True)
    mmax = jnp.max(mm, axis=1, keepdims=True)
    dmin = zsq - 2.0 * mmax
    db = lax.bitcast_convert_type(dmin, jnp.int32)
    dnext = lax.bitcast_convert_type(db + 1, jnp.float32)
    ulp = dnext - dmin
    h = ((zsq - dmin) - 0.5 * ulp) * 0.5
    hb = lax.bitcast_convert_type(h, jnp.int32)
    hup = lax.bitcast_convert_type(
        jnp.where(h >= 0.0, hb + 1, hb - 1), jnp.float32)
    thr = jnp.where((db & 1) == 0, h, hup)
    key = key_ref[...]
    sel = jnp.where(mm >= thr, key, jnp.float32(2.0 ** 24))
    kmin = jnp.min(sel, axis=1, keepdims=True)
    idx_ref[...] = lax.bitcast_convert_type(kmin, jnp.int32) - _KEY_BASE

    @pl.when(m == 0)
    def _init():
        acc_ref[0] = 0.0

    acc_ref[0] += jnp.sum(dmin)

    @pl.when(m == NSTEPS - 1)
    def _fin():
        scale = (1.0 + COMMIT) / (N_TOKENS * DIM)
        loss_ref[...] = jnp.broadcast_to(acc_ref[0] * scale, (1, 1))


def _distances_argmin(z, embedding_weight):
    keys = lax.bitcast_convert_type(
        jnp.arange(N_CODES, dtype=jnp.int32) + jnp.int32(_KEY_BASE),
        jnp.float32).reshape(1, N_CODES)
    return pl.pallas_call(
        _dist_argmin_body,
        grid=(NSTEPS,),
        in_specs=[
            pl.BlockSpec((BM, DIM), lambda m: (m, 0)),
            pl.BlockSpec((N_CODES, DIM), lambda m: (0, 0)),
            pl.BlockSpec((1, N_CODES), lambda m: (0, 0)),
        ],
        out_specs=[
            pl.BlockSpec((BM, 1), lambda m: (m, 0)),
            pl.BlockSpec((1, 1), lambda m: (0, 0)),
        ],
        out_shape=[
            jax.ShapeDtypeStruct((N_TOKENS, 1), jnp.int32),
            jax.ShapeDtypeStruct((1, 1), jnp.float32),
        ],
        scratch_shapes=[pltpu.SMEM((1,), jnp.float32)],
        compiler_params=pltpu.CompilerParams(
            dimension_semantics=("arbitrary",)),
    )(z, embedding_weight, keys)


def _sc_gather(table, idx):
    info = plsc.get_sparse_core_info()
    nw = info.num_cores * info.num_subcores
    b_per_w = N_TOKENS // nw
    ch = 128
    nch = b_per_w // ch
    mesh = plsc.VectorSubcoreMesh(core_axis_name="c", subcore_axis_name="s")

    @functools.partial(
        pl.kernel, mesh=mesh,
        out_type=jax.ShapeDtypeStruct((N_TOKENS, DIM), jnp.float32),
        scratch_types=[
            pltpu.VMEM((b_per_w,), jnp.int32),
            pltpu.VMEM((ch, DIM), jnp.float32),
            pltpu.VMEM((ch, DIM), jnp.float32),
            pltpu.SemaphoreType.DMA,
            pltpu.SemaphoreType.DMA,
        ],
    )
    def k(table_hbm, idx_hbm, out_hbm, idx_v, buf0, buf1, sem0, sem1):
        wid = lax.axis_index("s") * info.num_cores + lax.axis_index("c")
        base = wid * b_per_w
        pltpu.sync_copy(idx_hbm.at[pl.ds(base, b_per_w)], idx_v)
        bufs = (buf0, buf1)
        sems = (sem0, sem1)
        copies = [None, None]
        copies[0] = pltpu.async_copy(
            table_hbm.at[idx_v.at[pl.ds(0, ch)]], bufs[0], sems[0])
        for c in range(nch):
            if c + 1 < nch:
                copies[(c + 1) % 2] = pltpu.async_copy(
                    table_hbm.at[idx_v.at[pl.ds((c + 1) * ch, ch)]],
                    bufs[(c + 1) % 2], sems[(c + 1) % 2])
            copies[c % 2].wait()
            pltpu.sync_copy(bufs[c % 2], out_hbm.at[pl.ds(base + c * ch, ch)])

    return k(table, idx)


def kernel(z, embedding_weight):
    z_flat = z.reshape(N_TOKENS, DIM)
    idx2d, loss2d = _distances_argmin(z_flat, embedding_weight)
    code_indices = idx2d.reshape(N_TOKENS)
    quantized_st = _sc_gather(embedding_weight, code_indices)
    loss = loss2d.reshape(())
    return (quantized_st, loss, code_indices)


# speedup vs baseline: 1.5354x; 1.0042x over previous
import functools

import jax
import jax.numpy as jnp
from jax import lax
from jax.experimental import pallas as pl
from jax.experimental.pallas import tpu as pltpu
from jax.experimental.pallas import tpu_sc as plsc

N_TOKENS = 16384
N_CODES = 8192
DIM = 256
COMMIT = 0.25

BM = 1024
NSTEPS = N_TOKENS // BM

_KEY_BASE = 0x4B000000


def _dist_argmin_body(z_ref, e_ref, idx_ref, loss_ref, acc_ref):
    m = pl.program_id(0)
    zb = z_ref[...]
    eb = e_ref[...]
    mm = lax.dot_general(zb, eb, (((1,), (1,)), ((), ())),
                         preferred_element_type=jnp.float32)
    zsq = jnp.sum(zb * zb, axis=1, keepdims=True)
    mmax = jnp.max(mm, axis=1, keepdims=True)
    dmin = zsq - 2.0 * mmax
    db = lax.bitcast_convert_type(dmin, jnp.int32)
    dnext = lax.bitcast_convert_type(db + 1, jnp.float32)
    ulp = dnext - dmin
    h = ((zsq - dmin) - 0.5 * ulp) * 0.5
    hb = lax.bitcast_convert_type(h, jnp.int32)
    hup = lax.bitcast_convert_type(
        jnp.where(h >= 0.0, hb + 1, hb - 1), jnp.float32)
    thr = jnp.where((db & 1) == 0, h, hup)
    key = lax.bitcast_convert_type(
        lax.broadcasted_iota(jnp.int32, (1, N_CODES), 1) + _KEY_BASE,
        jnp.float32)
    sel = jnp.where(mm >= thr, key, jnp.float32(2.0 ** 24))
    kmin = jnp.min(sel, axis=1, keepdims=True)
    idx_ref[...] = lax.bitcast_convert_type(kmin, jnp.int32) - _KEY_BASE

    @pl.when(m == 0)
    def _init():
        acc_ref[0] = 0.0

    acc_ref[0] += jnp.sum(dmin)

    @pl.when(m == NSTEPS - 1)
    def _fin():
        scale = (1.0 + COMMIT) / (N_TOKENS * DIM)
        loss_ref[...] = jnp.broadcast_to(acc_ref[0] * scale, (1, 1))


def _distances_argmin(z, embedding_weight):
    return pl.pallas_call(
        _dist_argmin_body,
        grid=(NSTEPS,),
        in_specs=[
            pl.BlockSpec((BM, DIM), lambda m: (m, 0)),
            pl.BlockSpec((N_CODES, DIM), lambda m: (0, 0)),
        ],
        out_specs=[
            pl.BlockSpec((BM, 1), lambda m: (m, 0)),
            pl.BlockSpec((1, 1), lambda m: (0, 0)),
        ],
        out_shape=[
            jax.ShapeDtypeStruct((N_TOKENS, 1), jnp.int32),
            jax.ShapeDtypeStruct((1, 1), jnp.float32),
        ],
        scratch_shapes=[pltpu.SMEM((1,), jnp.float32)],
        compiler_params=pltpu.CompilerParams(
            dimension_semantics=("arbitrary",)),
    )(z, embedding_weight)


def _sc_gather(table, idx):
    info = plsc.get_sparse_core_info()
    nw = info.num_cores * info.num_subcores
    b_per_w = N_TOKENS // nw
    ch = 128
    nch = b_per_w // ch
    mesh = plsc.VectorSubcoreMesh(core_axis_name="c", subcore_axis_name="s")

    @functools.partial(
        pl.kernel, mesh=mesh,
        out_type=jax.ShapeDtypeStruct((N_TOKENS, DIM), jnp.float32),
        scratch_types=[
            pltpu.VMEM((b_per_w,), jnp.int32),
            pltpu.VMEM((ch, DIM), jnp.float32),
            pltpu.VMEM((ch, DIM), jnp.float32),
            pltpu.SemaphoreType.DMA,
            pltpu.SemaphoreType.DMA,
        ],
    )
    def k(table_hbm, idx_hbm, out_hbm, idx_v, buf0, buf1, sem0, sem1):
        wid = lax.axis_index("s") * info.num_cores + lax.axis_index("c")
        base = wid * b_per_w
        pltpu.sync_copy(idx_hbm.at[pl.ds(base, b_per_w)], idx_v)
        bufs = (buf0, buf1)
        sems = (sem0, sem1)
        copies = [None, None]
        copies[0] = pltpu.async_copy(
            table_hbm.at[idx_v.at[pl.ds(0, ch)]], bufs[0], sems[0])
        for c in range(nch):
            if c + 1 < nch:
                copies[(c + 1) % 2] = pltpu.async_copy(
                    table_hbm.at[idx_v.at[pl.ds((c + 1) * ch, ch)]],
                    bufs[(c + 1) % 2], sems[(c + 1) % 2])
            copies[c % 2].wait()
            pltpu.sync_copy(bufs[c % 2], out_hbm.at[pl.ds(base + c * ch, ch)])

    return k(table, idx)


def kernel(z, embedding_weight):
    z_flat = z.reshape(N_TOKENS, DIM)
    idx2d, loss2d = _distances_argmin(z_flat, embedding_weight)
    code_indices = idx2d.reshape(N_TOKENS)
    quantized_st = _sc_gather(embedding_weight, code_indices)
    loss = loss2d.reshape(())
    return (quantized_st, loss, code_indices)
